# baseline (speedup 1.0000x reference)
import functools

import jax
import jax.numpy as jnp
from jax import lax
from jax.experimental import pallas as pl
from jax.experimental.pallas import tpu as pltpu
from jax.experimental.pallas import tpu_sc as plsc

VOCAB = 1000000
DIM = 32
BATCH = 16384
NEG = 5

_INFO = plsc.get_sparse_core_info()
NUM_CORES = _INFO.num_cores
NUM_SUBCORES = _INFO.num_subcores
LANES = _INFO.num_lanes
NW = NUM_CORES * NUM_SUBCORES
CHUNK = BATCH // NW
NGROUP = CHUNK // LANES
NN = CHUNK * NEG
FLAT = DIM * VOCAB


def _body(tw, cw, nwT, tinT, toutT, pos_out, negT_out,
          idx_t, idx_c, idx_n, tt, tc, tn, pos_v, neg_v, sem):
    wid = lax.axis_index("s") * NUM_CORES + lax.axis_index("c")
    base = wid * CHUNK

    pltpu.sync_copy(tw.at[pl.ds(base, CHUNK)], idx_t)
    pltpu.sync_copy(cw.at[pl.ds(base, CHUNK)], idx_c)
    for k in range(NEG):
        pltpu.sync_copy(nwT.at[k, pl.ds(base, CHUNK)],
                        idx_n.at[pl.ds(k * CHUNK, CHUNK)])

    def gather(g, carry):
        i0 = pl.multiple_of(g * LANES, LANES)
        rt = idx_t[pl.ds(i0, LANES)]
        rc = idx_c[pl.ds(i0, LANES)]
        for j in range(DIM):
            pltpu.async_copy(tinT.at[j].at[rt],
                             tt.at[pl.ds(j * CHUNK + i0, LANES)], sem)
            pltpu.async_copy(toutT.at[j].at[rc],
                             tc.at[pl.ds(j * CHUNK + i0, LANES)], sem)
        for k in range(NEG):
            rn = idx_n[pl.ds(k * CHUNK + i0, LANES)]
            for j in range(DIM):
                pltpu.async_copy(
                    toutT.at[j].at[rn],
                    tn.at[pl.ds(j * NN + k * CHUNK + i0, LANES)], sem)
        return carry

    lax.fori_loop(0, NGROUP, gather, 0)

    pltpu.make_async_copy(tinT.at[0, pl.ds(0, DIM * CHUNK)], tt, sem).wait()
    pltpu.make_async_copy(tinT.at[0, pl.ds(0, DIM * CHUNK)], tc, sem).wait()
    pltpu.make_async_copy(tinT.at[0, pl.ds(0, DIM * NN)], tn, sem).wait()

    def compute(g, carry):
        i0 = pl.multiple_of(g * LANES, LANES)
        accp = jnp.zeros((LANES,), jnp.float32)
        accn = [jnp.zeros((LANES,), jnp.float32) for _ in range(NEG)]
        for j in range(DIM):
            t = tt[pl.ds(j * CHUNK + i0, LANES)]
            c = tc[pl.ds(j * CHUNK + i0, LANES)]
            accp = accp + t * c
            for k in range(NEG):
                n = tn[pl.ds(j * NN + k * CHUNK + i0, LANES)]
                accn[k] = accn[k] + t * n
        pos_v[pl.ds(i0, LANES)] = accp
        for k in range(NEG):
            neg_v[pl.ds(k * CHUNK + i0, LANES)] = accn[k]
        return carry

    lax.fori_loop(0, NGROUP, compute, 0)

    pltpu.sync_copy(pos_v, pos_out.at[pl.ds(base, CHUNK)])
    for k in range(NEG):
        pltpu.sync_copy(neg_v.at[pl.ds(k * CHUNK, CHUNK)],
                        negT_out.at[k, pl.ds(base, CHUNK)])


@jax.jit
def _run(tw, cw, nwT, tinT, toutT):
    f = pl.kernel(
        _body,
        out_type=[
            jax.ShapeDtypeStruct((BATCH,), jnp.float32),
            jax.ShapeDtypeStruct((NEG, BATCH), jnp.float32),
        ],
        mesh=plsc.VectorSubcoreMesh(core_axis_name="c", subcore_axis_name="s"),
        compiler_params=pltpu.CompilerParams(
            needs_layout_passes=False, use_tc_tiling_on_sc=False),
        scratch_types=[
            pltpu.VMEM((CHUNK,), jnp.int32),
            pltpu.VMEM((CHUNK,), jnp.int32),
            pltpu.VMEM((NEG * CHUNK,), jnp.int32),
            pltpu.VMEM((DIM * CHUNK,), jnp.float32),
            pltpu.VMEM((DIM * CHUNK,), jnp.float32),
            pltpu.VMEM((DIM * NN,), jnp.float32),
            pltpu.VMEM((CHUNK,), jnp.float32),
            pltpu.VMEM((NEG * CHUNK,), jnp.float32),
            pltpu.SemaphoreType.DMA,
        ],
    )
    return f(tw, cw, nwT, tinT, toutT)


def kernel(target_word, context_word, negative_words, emb_in, emb_out):
    tw = target_word.astype(jnp.int32)
    cw = context_word.astype(jnp.int32)
    nwT = negative_words.astype(jnp.int32).T
    tinT = emb_in.T
    toutT = emb_out.T
    pos, negT = _run(tw, cw, nwT, tinT, toutT)
    return (pos, negT.T)


# speedup vs baseline: 13.5118x; 13.5118x over previous
import functools

import jax
import jax.numpy as jnp
from jax import lax
from jax.experimental import pallas as pl
from jax.experimental.pallas import tpu as pltpu
from jax.experimental.pallas import tpu_sc as plsc

VOCAB = 1000000
DIM = 32
BATCH = 16384
NEG = 5

_INFO = plsc.get_sparse_core_info()
NUM_CORES = _INFO.num_cores
NUM_SUBCORES = _INFO.num_subcores
LANES = _INFO.num_lanes
NW = NUM_CORES * NUM_SUBCORES
CHUNK = BATCH // NW
NGROUP = CHUNK // LANES
NN = CHUNK * NEG
FLAT = DIM * VOCAB

CB = 512
BW = DIM * CB
NBLK = VOCAB // CB
NBW = -(-NBLK // NW)
REM0 = NBLK * CB
REM = VOCAB - REM0
ROWS = NBLK * 128
TAILROW = ROWS
FLAT2 = (ROWS + DIM) * 128


def _detile_body(tinT, toutT, tail_a, tail_b, tin_f, tout_f,
                 buf_a, buf_b, sem_a, sem_b):
    wid = lax.axis_index("s") * NUM_CORES + lax.axis_index("c")

    def run_table(src, tail, dst, buf, sem):
        def start(b):
            blk = wid + NW * b

            @pl.when(blk < NBLK)
            def _():
                c0 = pl.multiple_of(blk * CB, CB)
                pltpu.async_copy(src.at[:, pl.ds(c0, CB)],
                                 buf.at[b % 2], sem)

        start(0)

        def step(b, carry):
            blk = wid + NW * b

            @pl.when(blk < NBLK)
            def _():
                c0 = pl.multiple_of(blk * CB, CB)
                pltpu.make_async_copy(src.at[:, pl.ds(c0, CB)],
                                      buf.at[b % 2], sem).wait()

            @pl.when(b + 1 < NBW)
            def _():
                start(b + 1)

            @pl.when(blk < NBLK)
            def _():
                r0 = pl.multiple_of(blk * 128, 128)
                for p in range(CB // 128):
                    pltpu.sync_copy(
                        buf.at[b % 2, :, pl.ds(p * 128, 128)],
                        dst.at[pl.ds(r0 + p * DIM, DIM), :])
            return carry

        lax.fori_loop(0, NBW, step, 0, unroll=2)

        @pl.when(wid == 0)
        def _():
            pltpu.sync_copy(tail, buf.at[0, :, pl.ds(0, 128)])
            pltpu.sync_copy(buf.at[0, :, pl.ds(0, 128)],
                            dst.at[pl.ds(TAILROW, DIM), :])

    run_table(tinT, tail_a, tin_f, buf_a, sem_a)
    run_table(toutT, tail_b, tout_f, buf_b, sem_b)


def _score_body(tw, cw, nwT, tin_f, tout_f, pos_out, negT_out,
                idx_t, idx_c, idx_n, tt, tc, tn, pos_v, neg_v, sem):
    wid = lax.axis_index("s") * NUM_CORES + lax.axis_index("c")
    base = wid * CHUNK

    pltpu.sync_copy(tw.at[pl.ds(base, CHUNK)], idx_t)
    pltpu.sync_copy(cw.at[pl.ds(base, CHUNK)], idx_c)
    for k in range(NEG):
        pltpu.sync_copy(nwT.at[k, pl.ds(base, CHUNK)],
                        idx_n.at[pl.ds(k * CHUNK, CHUNK)])

    def flat_base(r):
        in_tail = r >= REM0
        full = ((r >> 7) << 12) + (r & 127)
        tail = r + (TAILROW * 128 - REM0)
        return jnp.where(in_tail, tail, full)

    def gather(g, carry):
        i0 = pl.multiple_of(g * LANES, LANES)
        at = flat_base(idx_t[pl.ds(i0, LANES)])
        ac = flat_base(idx_c[pl.ds(i0, LANES)])
        for j in range(DIM):
            pltpu.async_copy(tin_f.at[at + j * 128],
                             tt.at[pl.ds(j * CHUNK + i0, LANES)], sem)
            pltpu.async_copy(tout_f.at[ac + j * 128],
                             tc.at[pl.ds(j * CHUNK + i0, LANES)], sem)
        for k in range(NEG):
            an = flat_base(idx_n[pl.ds(k * CHUNK + i0, LANES)])
            for j in range(DIM):
                pltpu.async_copy(
                    tout_f.at[an + j * 128],
                    tn.at[pl.ds(j * NN + k * CHUNK + i0, LANES)], sem)
        return carry

    lax.fori_loop(0, NGROUP, gather, 0)

    pltpu.make_async_copy(tin_f.at[pl.ds(0, DIM * CHUNK)], tt, sem).wait()
    pltpu.make_async_copy(tin_f.at[pl.ds(0, DIM * CHUNK)], tc, sem).wait()
    pltpu.make_async_copy(tin_f.at[pl.ds(0, DIM * NN)], tn, sem).wait()

    def compute(g, carry):
        i0 = pl.multiple_of(g * LANES, LANES)
        accp = jnp.zeros((LANES,), jnp.float32)
        accn = [jnp.zeros((LANES,), jnp.float32) for _ in range(NEG)]
        for j in range(DIM):
            t = tt[pl.ds(j * CHUNK + i0, LANES)]
            c = tc[pl.ds(j * CHUNK + i0, LANES)]
            accp = accp + t * c
            for k in range(NEG):
                n = tn[pl.ds(j * NN + k * CHUNK + i0, LANES)]
                accn[k] = accn[k] + t * n
        pos_v[pl.ds(i0, LANES)] = accp
        for k in range(NEG):
            neg_v[pl.ds(k * CHUNK + i0, LANES)] = accn[k]
        return carry

    lax.fori_loop(0, NGROUP, compute, 0)

    pltpu.sync_copy(pos_v, pos_out.at[pl.ds(base, CHUNK)])
    for k in range(NEG):
        pltpu.sync_copy(neg_v.at[pl.ds(k * CHUNK, CHUNK)],
                        negT_out.at[k, pl.ds(base, CHUNK)])


@jax.jit
def _run(tw, cw, nwT, tinT, toutT, tail_a, tail_b):
    mesh = plsc.VectorSubcoreMesh(core_axis_name="c", subcore_axis_name="s")
    detile = pl.kernel(
        _detile_body,
        out_type=[
            jax.ShapeDtypeStruct((ROWS + DIM, 128), jnp.float32),
            jax.ShapeDtypeStruct((ROWS + DIM, 128), jnp.float32),
        ],
        mesh=mesh,
        compiler_params=pltpu.CompilerParams(
            needs_layout_passes=False, use_tc_tiling_on_sc=True),
        scratch_types=[
            pltpu.VMEM((2, DIM, CB), jnp.float32),
            pltpu.VMEM((2, DIM, CB), jnp.float32),
            pltpu.SemaphoreType.DMA,
            pltpu.SemaphoreType.DMA,
        ],
    )
    tin_f2, tout_f2 = detile(tinT, toutT, tail_a, tail_b)
    tin_f = tin_f2.reshape(FLAT2)
    tout_f = tout_f2.reshape(FLAT2)

    score = pl.kernel(
        _score_body,
        out_type=[
            jax.ShapeDtypeStruct((BATCH,), jnp.float32),
            jax.ShapeDtypeStruct((NEG, BATCH), jnp.float32),
        ],
        mesh=mesh,
        compiler_params=pltpu.CompilerParams(
            needs_layout_passes=False, use_tc_tiling_on_sc=False),
        scratch_types=[
            pltpu.VMEM((CHUNK,), jnp.int32),
            pltpu.VMEM((CHUNK,), jnp.int32),
            pltpu.VMEM((NEG * CHUNK,), jnp.int32),
            pltpu.VMEM((DIM * CHUNK,), jnp.float32),
            pltpu.VMEM((DIM * CHUNK,), jnp.float32),
            pltpu.VMEM((DIM * NN,), jnp.float32),
            pltpu.VMEM((CHUNK,), jnp.float32),
            pltpu.VMEM((NEG * CHUNK,), jnp.float32),
            pltpu.SemaphoreType.DMA,
        ],
    )
    return score(tw, cw, nwT, tin_f, tout_f)


def kernel(target_word, context_word, negative_words, emb_in, emb_out):
    tw = target_word.astype(jnp.int32)
    cw = context_word.astype(jnp.int32)
    nwT = negative_words.astype(jnp.int32).T
    tinT = emb_in.T
    toutT = emb_out.T
    pad = ((0, 0), (0, 128 - REM))
    tail_a = jnp.pad(emb_in[REM0:].T, pad)
    tail_b = jnp.pad(emb_out[REM0:].T, pad)
    pos, negT = _run(tw, cw, nwT, tinT, toutT, tail_a, tail_b)
    return (pos, negT.T)
